# Initial kernel scaffold; baseline (speedup 1.0000x reference)
import jax
import jax.numpy as jnp
from jax.experimental import pallas as pl


def kernel(x, embedding_matrix):
    raise NotImplementedError("write your pallas kernel here")


# speedup vs baseline: 1.7975x; 1.7975x over previous
import functools

import jax
import jax.numpy as jnp
from jax import lax
from jax.experimental import pallas as pl
from jax.experimental.pallas import tpu as pltpu
from jax.experimental.pallas import tpu_sc as plsc

_B, _S = 16384, 50
_D = 64
_TOTAL = _B * _S
_NC, _NS = 2, 16
_NW = _NC * _NS
_PER_W = _TOTAL // _NW
_CHUNK = 512
_N_CHUNKS = _PER_W // _CHUNK


def _body(table_hbm, idx_hbm, out_hbm, idx_v, rows_v, sem):
    wid = lax.axis_index("s") * _NC + lax.axis_index("c")
    base = wid * _PER_W

    def step(i, _):
        off = pl.multiple_of(base + i * _CHUNK, _CHUNK)
        pltpu.sync_copy(idx_hbm.at[pl.ds(off, _CHUNK)], idx_v)
        pltpu.async_copy(table_hbm.at[idx_v], rows_v, sem).wait()
        pltpu.sync_copy(rows_v, out_hbm.at[pl.ds(off, _CHUNK)])
        return 0

    lax.fori_loop(0, _N_CHUNKS, step, 0)


@jax.jit
def _embed(x_flat, table):
    mesh = plsc.VectorSubcoreMesh(
        core_axis_name="c", subcore_axis_name="s",
        num_cores=_NC, num_subcores=_NS,
    )
    fn = pl.kernel(
        _body,
        out_type=jax.ShapeDtypeStruct((_TOTAL, _D), jnp.float32),
        mesh=mesh,
        scratch_types=[
            pltpu.VMEM((_CHUNK,), jnp.int32),
            pltpu.VMEM((_CHUNK, _D), jnp.float32),
            pltpu.SemaphoreType.DMA,
        ],
        compiler_params=pltpu.CompilerParams(use_tc_tiling_on_sc=False),
    )
    return fn(table, x_flat)


def kernel(x, embedding_matrix):
    x_flat = x.reshape(-1).astype(jnp.int32)
    out = _embed(x_flat, embedding_matrix)
    return out.reshape(_B, _S, _D)


# speedup vs baseline: 1.8750x; 1.0431x over previous
import jax
import jax.numpy as jnp
from jax import lax
from jax.experimental import pallas as pl
from jax.experimental.pallas import tpu as pltpu
from jax.experimental.pallas import tpu_sc as plsc

_B, _S = 16384, 50
_D = 64
_TOTAL = _B * _S
_NC, _NS = 2, 16
_NW = _NC * _NS
_PER_W = _TOTAL // _NW
_CHUNK = 512
_N_CHUNKS = _PER_W // _CHUNK


def _body(table_hbm, idx_hbm, out_hbm, idx_v, rows_v, sem_g, sem_o):
    wid = lax.axis_index("s") * _NC + lax.axis_index("c")
    base = wid * _PER_W

    pltpu.sync_copy(idx_hbm.at[pl.ds(base, _PER_W)], idx_v)

    def gather_desc(i, b):
        off = pl.multiple_of(i * _CHUNK, _CHUNK)
        return pltpu.make_async_copy(
            table_hbm.at[idx_v.at[pl.ds(off, _CHUNK)]], rows_v.at[b], sem_g.at[b]
        )

    def write_desc(i, b):
        off = pl.multiple_of(base + i * _CHUNK, _CHUNK)
        return pltpu.make_async_copy(
            rows_v.at[b], out_hbm.at[pl.ds(off, _CHUNK)], sem_o.at[b]
        )

    gather_desc(0, 0).start()
    gather_desc(1, 1).start()

    def half_step(i, b):
        gather_desc(i, b).wait()
        write_desc(i, b).start()

        @pl.when(i + 2 < _N_CHUNKS)
        def _():
            write_desc(i, b).wait()
            gather_desc(i + 2, b).start()

    def step(j, _):
        half_step(2 * j, 0)
        half_step(2 * j + 1, 1)
        return 0

    lax.fori_loop(0, _N_CHUNKS // 2, step, 0)
    write_desc(_N_CHUNKS - 2, 0).wait()
    write_desc(_N_CHUNKS - 1, 1).wait()


@jax.jit
def _embed(x_flat, table):
    mesh = plsc.VectorSubcoreMesh(
        core_axis_name="c", subcore_axis_name="s",
        num_cores=_NC, num_subcores=_NS,
    )
    fn = pl.kernel(
        _body,
        out_type=jax.ShapeDtypeStruct((_TOTAL, _D), jnp.float32),
        mesh=mesh,
        scratch_types=[
            pltpu.VMEM((_PER_W,), jnp.int32),
            pltpu.VMEM((2, _CHUNK, _D), jnp.float32),
            pltpu.SemaphoreType.DMA((2,)),
            pltpu.SemaphoreType.DMA((2,)),
        ],
        compiler_params=pltpu.CompilerParams(use_tc_tiling_on_sc=False),
    )
    return fn(table, x_flat)


def kernel(x, embedding_matrix):
    x_flat = x.reshape(-1).astype(jnp.int32)
    out = _embed(x_flat, embedding_matrix)
    return out.reshape(_B, _S, _D)


# speedup vs baseline: 1.8769x; 1.0011x over previous
import jax
import jax.numpy as jnp
from jax import lax
from jax.experimental import pallas as pl
from jax.experimental.pallas import tpu as pltpu
from jax.experimental.pallas import tpu_sc as plsc

_B, _S = 16384, 50
_D = 64
_TOTAL = _B * _S
_NC, _NS = 2, 16
_NW = _NC * _NS
_PER_W = _TOTAL // _NW
_BPC = 8
_CHUNK = _BPC * _S
_N_CHUNKS = _PER_W // _CHUNK
_B_PER_W = _B // _NW


def _body(table_hbm, idx_hbm, out_hbm, idx_v, rows_v, sem_g, sem_o):
    wid = lax.axis_index("s") * _NC + lax.axis_index("c")
    base = wid * _PER_W

    pltpu.sync_copy(idx_hbm.at[pl.ds(base, _PER_W)], idx_v)

    def gather_desc(i, b):
        off = pl.multiple_of(i * _CHUNK, _CHUNK)
        return pltpu.make_async_copy(
            table_hbm.at[idx_v.at[pl.ds(off, _CHUNK)]], rows_v.at[b], sem_g.at[b]
        )

    def write_start(i, b):
        wb0 = wid * _B_PER_W
        for k in range(_BPC):
            pltpu.make_async_copy(
                rows_v.at[b, pl.ds(k * _S, _S)],
                out_hbm.at[wb0 + i * _BPC + k],
                sem_o.at[b],
            ).start()

    def write_wait(i, b):
        for k in range(_BPC):
            pltpu.make_async_copy(
                rows_v.at[b, pl.ds(k * _S, _S)],
                out_hbm.at[wid * _B_PER_W + i * _BPC + k],
                sem_o.at[b],
            ).wait()

    gather_desc(0, 0).start()
    gather_desc(1, 1).start()

    def half_step(i, b):
        gather_desc(i, b).wait()
        write_start(i, b)

        @pl.when(i + 2 < _N_CHUNKS)
        def _():
            write_wait(i, b)
            gather_desc(i + 2, b).start()

    def step(j, _):
        half_step(2 * j, 0)
        half_step(2 * j + 1, 1)
        return 0

    lax.fori_loop(0, _N_CHUNKS // 2, step, 0)
    write_wait(_N_CHUNKS - 2, 0)
    write_wait(_N_CHUNKS - 1, 1)


@jax.jit
def _embed(x_flat, table):
    mesh = plsc.VectorSubcoreMesh(
        core_axis_name="c", subcore_axis_name="s",
        num_cores=_NC, num_subcores=_NS,
    )
    fn = pl.kernel(
        _body,
        out_type=jax.ShapeDtypeStruct((_B, _S, _D), jnp.float32),
        mesh=mesh,
        scratch_types=[
            pltpu.VMEM((_PER_W,), jnp.int32),
            pltpu.VMEM((2, _CHUNK, _D), jnp.float32),
            pltpu.SemaphoreType.DMA((2,)),
            pltpu.SemaphoreType.DMA((2,)),
        ],
        compiler_params=pltpu.CompilerParams(use_tc_tiling_on_sc=False),
    )
    return fn(table, x_flat)


def kernel(x, embedding_matrix):
    x_flat = x.reshape(-1).astype(jnp.int32)
    return _embed(x_flat, embedding_matrix)
